# baseline (speedup 1.0000x reference)
import functools

import jax
import jax.numpy as jnp
from jax import lax
from jax.experimental import pallas as pl
from jax.experimental.pallas import tpu as pltpu
from jax.experimental.pallas import tpu_sc as plsc

EPS = 1e-5
GW = 128
TB = 512
def _transpose_body(x_ref, o_ref):
    d, v = x_ref.shape[1], x_ref.shape[2]
    ch = 2000
    for i in range(v // ch):
        x = x_ref[0, :, pl.ds(i * ch, ch)]
        x3 = x.reshape(d, ch // 8, 8)
        y3 = jnp.transpose(x3, (1, 2, 0))
        o_ref[0, pl.ds(i * (ch // 8), ch // 8), :] = y3.reshape(ch // 8, 8 * d)


def _tc_relayout_tables(tables):
    f, v, d = tables.shape
    vg = v // 8
    vgp = ((vg + 7) // 8) * 8
    t_t = jnp.transpose(tables, (0, 2, 1))
    t128 = pl.pallas_call(
        _transpose_body,
        grid=(f,),
        in_specs=[pl.BlockSpec((1, d, v), lambda i: (i, 0, 0))],
        out_specs=pl.BlockSpec((1, vgp, 8 * d), lambda i: (i, 0, 0)),
        out_shape=jax.ShapeDtypeStruct((f, vgp, 8 * d), jnp.float32),
    )(t_t)
    return t128.reshape(f * vgp * 8, d), vgp * 8


def _sc_gather(tables_flat, idx2d):
    n_idx = idx2d.shape[1]
    d = tables_flat.shape[1]
    mesh = plsc.VectorSubcoreMesh(core_axis_name="core", subcore_axis_name="subcore")

    @functools.partial(
        pl.kernel,
        out_type=jax.ShapeDtypeStruct((n_idx, d), tables_flat.dtype),
        mesh=mesh,
        compiler_params=pltpu.CompilerParams(use_tc_tiling_on_sc=False),
    )
    def gather_kernel(tab_hbm, idx_hbm, out_hbm):
        def body(i_vmem, o_vmem):
            pltpu.sync_copy(tab_hbm.at[i_vmem.at[0]], o_vmem)

        pltpu.emit_pipeline(
            body,
            grid=(n_idx // GW,),
            in_specs=[pl.BlockSpec((1, GW), index_map=lambda i: (0, i))],
            out_specs=[pl.BlockSpec((GW, d), index_map=lambda i: (i, 0))],
            core_axis_name=("core", "subcore"),
            dimension_semantics=(pltpu.PARALLEL,),
        )(idx_hbm, out_hbm)

    return gather_kernel(tables_flat, idx2d)


def _stats_update(s_ref, h):
    st = jnp.concatenate(
        [jnp.sum(h, axis=0, keepdims=True), jnp.sum(h * h, axis=0, keepdims=True)], axis=0
    )

    @pl.when(pl.program_id(0) == 0)
    def _():
        s_ref[...] = st

    @pl.when(pl.program_id(0) != 0)
    def _():
        s_ref[...] += st


def _layer1_body(num_ref, emb_ref, wn_ref, we_ref, b_ref, h_ref, s_ref):
    h = jnp.dot(num_ref[...], wn_ref[...], preferred_element_type=jnp.float32)
    h = h + jnp.dot(emb_ref[...], we_ref[...], preferred_element_type=jnp.float32)
    h = h + b_ref[...]
    h_ref[...] = h
    _stats_update(s_ref, h)


def _bn_relu(h, s, g, be, batch):
    mu = s[0:1, :] * (1.0 / batch)
    var = s[1:2, :] * (1.0 / batch) - mu * mu
    scale = g * lax.rsqrt(var + EPS)
    shift = be - mu * scale
    return jnp.maximum(h * scale + shift, 0.0)


def _mid_body(h_ref, s_ref, g_ref, be_ref, w_ref, b_ref, o_ref, so_ref, *, batch):
    a = _bn_relu(h_ref[...], s_ref[...], g_ref[...], be_ref[...], batch)
    h = jnp.dot(a, w_ref[...], preferred_element_type=jnp.float32) + b_ref[...]
    o_ref[...] = h
    _stats_update(so_ref, h)


def _last_body(h_ref, s_ref, g_ref, be_ref, w_ref, b_ref, o_ref, *, batch):
    a = _bn_relu(h_ref[...], s_ref[...], g_ref[...], be_ref[...], batch)
    o_ref[...] = jnp.dot(a, w_ref[...], preferred_element_type=jnp.float32) + b_ref[...]


def _full(shape):
    return pl.BlockSpec(shape, lambda i: (0,) * len(shape))


def _layer1(num, emb, wn, we, b):
    batch, h1 = num.shape[0], wn.shape[1]
    return pl.pallas_call(
        _layer1_body,
        grid=(batch // TB,),
        in_specs=[
            pl.BlockSpec((TB, num.shape[1]), lambda i: (i, 0)),
            pl.BlockSpec((TB, emb.shape[1]), lambda i: (i, 0)),
            _full(wn.shape),
            _full(we.shape),
            _full(b.shape),
        ],
        out_specs=[
            pl.BlockSpec((TB, h1), lambda i: (i, 0)),
            _full((2, h1)),
        ],
        out_shape=[
            jax.ShapeDtypeStruct((batch, h1), jnp.float32),
            jax.ShapeDtypeStruct((2, h1), jnp.float32),
        ],
    )(num, emb, wn, we, b)


def _mid(h, s, g, be, w, b):
    batch, hout = h.shape[0], w.shape[1]
    return pl.pallas_call(
        functools.partial(_mid_body, batch=batch),
        grid=(batch // TB,),
        in_specs=[
            pl.BlockSpec((TB, h.shape[1]), lambda i: (i, 0)),
            _full(s.shape),
            _full(g.shape),
            _full(be.shape),
            _full(w.shape),
            _full(b.shape),
        ],
        out_specs=[
            pl.BlockSpec((TB, hout), lambda i: (i, 0)),
            _full((2, hout)),
        ],
        out_shape=[
            jax.ShapeDtypeStruct((batch, hout), jnp.float32),
            jax.ShapeDtypeStruct((2, hout), jnp.float32),
        ],
    )(h, s, g, be, w, b)


def _last(h, s, g, be, w, b):
    batch, hout = h.shape[0], w.shape[1]
    return pl.pallas_call(
        functools.partial(_last_body, batch=batch),
        grid=(batch // TB,),
        in_specs=[
            pl.BlockSpec((TB, h.shape[1]), lambda i: (i, 0)),
            _full(s.shape),
            _full(g.shape),
            _full(be.shape),
            _full(w.shape),
            _full(b.shape),
        ],
        out_specs=pl.BlockSpec((TB, hout), lambda i: (i, 0)),
        out_shape=jax.ShapeDtypeStruct((batch, hout), jnp.float32),
    )(h, s, g, be, w, b)


def kernel(numeric_features, cat_features, tables,
           W1, b1, g1, be1, W2, b2, g2, be2, W3, b3, g3, be3, W4, b4):
    batch, num_dim = numeric_features.shape
    f, v, d = tables.shape
    h1_dim = W1.shape[1]

    table_rows, stride = _tc_relayout_tables(tables)

    slots = 32
    idx = cat_features + (jnp.arange(f, dtype=jnp.int32) * stride)[None, :]
    idx = jnp.pad(idx, ((0, 0), (0, slots - f)))
    emb = _sc_gather(table_rows, idx.reshape(1, batch * slots))
    emb = emb.reshape(batch, slots * d)

    w1e = jnp.concatenate(
        [W1[num_dim:], jnp.zeros((slots * d - f * d, h1_dim), jnp.float32)], axis=0
    )
    row = lambda x: x.reshape(1, -1)
    h1, s1 = _layer1(numeric_features, emb, W1[:num_dim], w1e, row(b1))
    h2, s2 = _mid(h1, s1, row(g1), row(be1), W2, row(b2))
    h3, s3 = _mid(h2, s2, row(g2), row(be2), W3, row(b3))
    return _last(h3, s3, row(g3), row(be3), W4, row(b4))


# speedup vs baseline: 4.8441x; 4.8441x over previous
import functools

import jax
import jax.numpy as jnp
from jax import lax
from jax.experimental import pallas as pl
from jax.experimental.pallas import tpu as pltpu
from jax.experimental.pallas import tpu_sc as plsc

EPS = 1e-5
GW = 128
TB = 512
def _transpose_body(x_ref, o_ref):
    d, v = x_ref.shape[1], x_ref.shape[2]
    lanes = 8 * d
    rep = jnp.where(
        lax.broadcasted_iota(jnp.int32, (d, lanes), 1) % d
        == lax.broadcasted_iota(jnp.int32, (d, lanes), 0),
        1.0, 0.0)
    ch = 2000
    keep = (lax.broadcasted_iota(jnp.int32, (ch, lanes), 1) // d
            == lax.broadcasted_iota(jnp.int32, (ch, lanes), 0) % 8)
    for i in range(v // ch):
        x = x_ref[0, :, pl.ds(i * ch, ch)]
        z = lax.dot_general(x, rep, (((0,), (0,)), ((), ())),
                            preferred_element_type=jnp.float32)
        zm = jnp.where(keep, z, 0.0)
        o_ref[0, pl.ds(i * (ch // 8), ch // 8), :] = jnp.sum(
            zm.reshape(ch // 8, 8, lanes), axis=1)


def _tc_relayout_tables(tables):
    f, v, d = tables.shape
    vg = v // 8
    vgp = ((vg + 7) // 8) * 8
    t_t = jnp.transpose(tables, (0, 2, 1))
    t128 = pl.pallas_call(
        _transpose_body,
        grid=(f,),
        in_specs=[pl.BlockSpec((1, d, v), lambda i: (i, 0, 0))],
        out_specs=pl.BlockSpec((1, vgp, 8 * d), lambda i: (i, 0, 0)),
        out_shape=jax.ShapeDtypeStruct((f, vgp, 8 * d), jnp.float32),
    )(t_t)
    return t128.reshape(f * vgp * 8, d), vgp * 8


def _sc_gather(tables_flat, idx2d):
    n_idx = idx2d.shape[1]
    d = tables_flat.shape[1]
    mesh = plsc.VectorSubcoreMesh(core_axis_name="core", subcore_axis_name="subcore")

    @functools.partial(
        pl.kernel,
        out_type=jax.ShapeDtypeStruct((n_idx, d), tables_flat.dtype),
        mesh=mesh,
        compiler_params=pltpu.CompilerParams(use_tc_tiling_on_sc=False),
    )
    def gather_kernel(tab_hbm, idx_hbm, out_hbm):
        def body(i_vmem, o_vmem):
            pltpu.sync_copy(tab_hbm.at[i_vmem.at[0]], o_vmem)

        pltpu.emit_pipeline(
            body,
            grid=(n_idx // GW,),
            in_specs=[pl.BlockSpec((1, GW), index_map=lambda i: (0, i))],
            out_specs=[pl.BlockSpec((GW, d), index_map=lambda i: (i, 0))],
            core_axis_name=("core", "subcore"),
            dimension_semantics=(pltpu.PARALLEL,),
        )(idx_hbm, out_hbm)

    return gather_kernel(tables_flat, idx2d)


def _stats_update(s_ref, h):
    st = jnp.concatenate(
        [jnp.sum(h, axis=0, keepdims=True), jnp.sum(h * h, axis=0, keepdims=True)], axis=0
    )

    @pl.when(pl.program_id(0) == 0)
    def _():
        s_ref[...] = st

    @pl.when(pl.program_id(0) != 0)
    def _():
        s_ref[...] += st


def _layer1_body(num_ref, emb_ref, wn_ref, we_ref, b_ref, h_ref, s_ref):
    h = jnp.dot(num_ref[...], wn_ref[...], preferred_element_type=jnp.float32)
    h = h + jnp.dot(emb_ref[...], we_ref[...], preferred_element_type=jnp.float32)
    h = h + b_ref[...]
    h_ref[...] = h
    _stats_update(s_ref, h)


def _bn_relu(h, s, g, be, batch):
    mu = s[0:1, :] * (1.0 / batch)
    var = s[1:2, :] * (1.0 / batch) - mu * mu
    scale = g * lax.rsqrt(var + EPS)
    shift = be - mu * scale
    return jnp.maximum(h * scale + shift, 0.0)


def _mid_body(h_ref, s_ref, g_ref, be_ref, w_ref, b_ref, o_ref, so_ref, *, batch):
    a = _bn_relu(h_ref[...], s_ref[...], g_ref[...], be_ref[...], batch)
    h = jnp.dot(a, w_ref[...], preferred_element_type=jnp.float32) + b_ref[...]
    o_ref[...] = h
    _stats_update(so_ref, h)


def _last_body(h_ref, s_ref, g_ref, be_ref, w_ref, b_ref, o_ref, *, batch):
    a = _bn_relu(h_ref[...], s_ref[...], g_ref[...], be_ref[...], batch)
    o_ref[...] = jnp.dot(a, w_ref[...], preferred_element_type=jnp.float32) + b_ref[...]


def _full(shape):
    return pl.BlockSpec(shape, lambda i: (0,) * len(shape))


def _layer1(num, emb, wn, we, b):
    batch, h1 = num.shape[0], wn.shape[1]
    return pl.pallas_call(
        _layer1_body,
        grid=(batch // TB,),
        in_specs=[
            pl.BlockSpec((TB, num.shape[1]), lambda i: (i, 0)),
            pl.BlockSpec((TB, emb.shape[1]), lambda i: (i, 0)),
            _full(wn.shape),
            _full(we.shape),
            _full(b.shape),
        ],
        out_specs=[
            pl.BlockSpec((TB, h1), lambda i: (i, 0)),
            _full((2, h1)),
        ],
        out_shape=[
            jax.ShapeDtypeStruct((batch, h1), jnp.float32),
            jax.ShapeDtypeStruct((2, h1), jnp.float32),
        ],
    )(num, emb, wn, we, b)


def _mid(h, s, g, be, w, b):
    batch, hout = h.shape[0], w.shape[1]
    return pl.pallas_call(
        functools.partial(_mid_body, batch=batch),
        grid=(batch // TB,),
        in_specs=[
            pl.BlockSpec((TB, h.shape[1]), lambda i: (i, 0)),
            _full(s.shape),
            _full(g.shape),
            _full(be.shape),
            _full(w.shape),
            _full(b.shape),
        ],
        out_specs=[
            pl.BlockSpec((TB, hout), lambda i: (i, 0)),
            _full((2, hout)),
        ],
        out_shape=[
            jax.ShapeDtypeStruct((batch, hout), jnp.float32),
            jax.ShapeDtypeStruct((2, hout), jnp.float32),
        ],
    )(h, s, g, be, w, b)


def _last(h, s, g, be, w, b):
    batch, hout = h.shape[0], w.shape[1]
    return pl.pallas_call(
        functools.partial(_last_body, batch=batch),
        grid=(batch // TB,),
        in_specs=[
            pl.BlockSpec((TB, h.shape[1]), lambda i: (i, 0)),
            _full(s.shape),
            _full(g.shape),
            _full(be.shape),
            _full(w.shape),
            _full(b.shape),
        ],
        out_specs=pl.BlockSpec((TB, hout), lambda i: (i, 0)),
        out_shape=jax.ShapeDtypeStruct((batch, hout), jnp.float32),
    )(h, s, g, be, w, b)


def kernel(numeric_features, cat_features, tables,
           W1, b1, g1, be1, W2, b2, g2, be2, W3, b3, g3, be3, W4, b4):
    batch, num_dim = numeric_features.shape
    f, v, d = tables.shape
    h1_dim = W1.shape[1]

    table_rows, stride = _tc_relayout_tables(tables)

    slots = 32
    idx = cat_features + (jnp.arange(f, dtype=jnp.int32) * stride)[None, :]
    idx = jnp.concatenate([idx, idx[:, : slots - f]], axis=1)
    emb = _sc_gather(table_rows, idx.reshape(1, batch * slots))
    emb = emb.reshape(batch, slots * d)

    w1e = jnp.concatenate(
        [W1[num_dim:], jnp.zeros((slots * d - f * d, h1_dim), jnp.float32)], axis=0
    )
    row = lambda x: x.reshape(1, -1)
    h1, s1 = _layer1(numeric_features, emb, W1[:num_dim], w1e, row(b1))
    h2, s2 = _mid(h1, s1, row(g1), row(be1), W2, row(b2))
    h3, s3 = _mid(h2, s2, row(g2), row(be2), W3, row(b3))
    return _last(h3, s3, row(g3), row(be3), W4, row(b4))


# speedup vs baseline: 5.2238x; 1.0784x over previous
import functools

import jax
import jax.numpy as jnp
from jax import lax
from jax.experimental import pallas as pl
from jax.experimental.pallas import tpu as pltpu
from jax.experimental.pallas import tpu_sc as plsc

EPS = 1e-5
GW = 128
TB = 512
def _transpose_body(x_ref, o_ref):
    d, v = x_ref.shape[1], x_ref.shape[2]
    lanes = 8 * d
    rep = jnp.where(
        lax.broadcasted_iota(jnp.int32, (d, lanes), 1) % d
        == lax.broadcasted_iota(jnp.int32, (d, lanes), 0),
        1.0, 0.0)
    ch = 2048
    chunks = [(i * ch, ch) for i in range(v // ch)]
    if v % ch:
        chunks.append((v - v % ch, v % ch))
    keep = {}
    for off, c in chunks:
        if c not in keep:
            keep[c] = (lax.broadcasted_iota(jnp.int32, (c, lanes), 1) // d
                       == lax.broadcasted_iota(jnp.int32, (c, lanes), 0) % 8)
        x = x_ref[0, :, pl.ds(off, c)]
        z = lax.dot_general(x, rep, (((0,), (0,)), ((), ())),
                            preferred_element_type=jnp.float32)
        zm = jnp.where(keep[c], z, 0.0)
        o_ref[0, pl.ds(off // 8, c // 8), :] = jnp.sum(
            zm.reshape(c // 8, 8, lanes), axis=1)


def _tc_relayout_tables(tables):
    f, v, d = tables.shape
    vg = v // 8
    vgp = ((vg + 7) // 8) * 8
    t_t = jnp.transpose(tables, (0, 2, 1))
    t128 = pl.pallas_call(
        _transpose_body,
        grid=(f,),
        in_specs=[pl.BlockSpec((1, d, v), lambda i: (i, 0, 0))],
        out_specs=pl.BlockSpec((1, vgp, 8 * d), lambda i: (i, 0, 0)),
        out_shape=jax.ShapeDtypeStruct((f, vgp, 8 * d), jnp.float32),
    )(t_t)
    return t128.reshape(f * vgp * 8, d), vgp * 8


def _sc_gather(tables_flat, idx2d):
    n_idx = idx2d.shape[1]
    d = tables_flat.shape[1]
    mesh = plsc.VectorSubcoreMesh(core_axis_name="core", subcore_axis_name="subcore")

    @functools.partial(
        pl.kernel,
        out_type=jax.ShapeDtypeStruct((n_idx, d), tables_flat.dtype),
        mesh=mesh,
        compiler_params=pltpu.CompilerParams(use_tc_tiling_on_sc=False),
    )
    def gather_kernel(tab_hbm, idx_hbm, out_hbm):
        def body(i_vmem, o_vmem):
            pltpu.sync_copy(tab_hbm.at[i_vmem.at[0]], o_vmem)

        pltpu.emit_pipeline(
            body,
            grid=(n_idx // GW,),
            in_specs=[pl.BlockSpec((1, GW), index_map=lambda i: (0, i))],
            out_specs=[pl.BlockSpec((GW, d), index_map=lambda i: (i, 0))],
            core_axis_name=("core", "subcore"),
            dimension_semantics=(pltpu.PARALLEL,),
        )(idx_hbm, out_hbm)

    return gather_kernel(tables_flat, idx2d)


def _stats_update(s_ref, h):
    st = jnp.concatenate(
        [jnp.sum(h, axis=0, keepdims=True), jnp.sum(h * h, axis=0, keepdims=True)], axis=0
    )

    @pl.when(pl.program_id(0) == 0)
    def _():
        s_ref[...] = st

    @pl.when(pl.program_id(0) != 0)
    def _():
        s_ref[...] += st


def _bdot(a, w):
    return jnp.dot(a, w, preferred_element_type=jnp.float32)


def _layer1_body(num_ref, emb_ref, wn_ref, we_ref, b_ref, h_ref, s_ref):
    h = _bdot(num_ref[...], wn_ref[...])
    h = h + _bdot(emb_ref[...], we_ref[...])
    h = h + b_ref[...]
    h_ref[...] = h
    _stats_update(s_ref, h)


def _bn_relu(h, s, g, be, batch):
    mu = s[0:1, :] * (1.0 / batch)
    var = s[1:2, :] * (1.0 / batch) - mu * mu
    scale = g * lax.rsqrt(var + EPS)
    shift = be - mu * scale
    return jnp.maximum(h * scale + shift, 0.0)


def _mid_body(h_ref, s_ref, g_ref, be_ref, w_ref, b_ref, o_ref, so_ref, *, batch):
    a = _bn_relu(h_ref[...], s_ref[...], g_ref[...], be_ref[...], batch)
    h = _bdot(a, w_ref[...]) + b_ref[...]
    o_ref[...] = h
    _stats_update(so_ref, h)


def _last_body(h_ref, s_ref, g_ref, be_ref, w_ref, b_ref, o_ref, *, batch):
    a = _bn_relu(h_ref[...], s_ref[...], g_ref[...], be_ref[...], batch)
    o_ref[...] = _bdot(a, w_ref[...]) + b_ref[...]


def _full(shape):
    return pl.BlockSpec(shape, lambda i: (0,) * len(shape))


def _layer1(num, emb, wn, we, b):
    batch, h1 = num.shape[0], wn.shape[1]
    return pl.pallas_call(
        _layer1_body,
        grid=(batch // TB,),
        in_specs=[
            pl.BlockSpec((TB, num.shape[1]), lambda i: (i, 0)),
            pl.BlockSpec((TB, emb.shape[1]), lambda i: (i, 0)),
            _full(wn.shape),
            _full(we.shape),
            _full(b.shape),
        ],
        out_specs=[
            pl.BlockSpec((TB, h1), lambda i: (i, 0)),
            _full((2, h1)),
        ],
        out_shape=[
            jax.ShapeDtypeStruct((batch, h1), jnp.float32),
            jax.ShapeDtypeStruct((2, h1), jnp.float32),
        ],
    )(num, emb, wn, we, b)


def _mid(h, s, g, be, w, b):
    batch, hout = h.shape[0], w.shape[1]
    return pl.pallas_call(
        functools.partial(_mid_body, batch=batch),
        grid=(batch // TB,),
        in_specs=[
            pl.BlockSpec((TB, h.shape[1]), lambda i: (i, 0)),
            _full(s.shape),
            _full(g.shape),
            _full(be.shape),
            _full(w.shape),
            _full(b.shape),
        ],
        out_specs=[
            pl.BlockSpec((TB, hout), lambda i: (i, 0)),
            _full((2, hout)),
        ],
        out_shape=[
            jax.ShapeDtypeStruct((batch, hout), jnp.float32),
            jax.ShapeDtypeStruct((2, hout), jnp.float32),
        ],
    )(h, s, g, be, w, b)


def _last(h, s, g, be, w, b):
    batch, hout = h.shape[0], w.shape[1]
    return pl.pallas_call(
        functools.partial(_last_body, batch=batch),
        grid=(batch // TB,),
        in_specs=[
            pl.BlockSpec((TB, h.shape[1]), lambda i: (i, 0)),
            _full(s.shape),
            _full(g.shape),
            _full(be.shape),
            _full(w.shape),
            _full(b.shape),
        ],
        out_specs=pl.BlockSpec((TB, hout), lambda i: (i, 0)),
        out_shape=jax.ShapeDtypeStruct((batch, hout), jnp.float32),
    )(h, s, g, be, w, b)


def kernel(numeric_features, cat_features, tables,
           W1, b1, g1, be1, W2, b2, g2, be2, W3, b3, g3, be3, W4, b4):
    batch, num_dim = numeric_features.shape
    f, v, d = tables.shape
    h1_dim = W1.shape[1]

    table_rows, stride = _tc_relayout_tables(tables)

    slots = 32
    idx = cat_features + (jnp.arange(f, dtype=jnp.int32) * stride)[None, :]
    idx = jnp.concatenate([idx, idx[:, : slots - f]], axis=1)
    emb = _sc_gather(table_rows, idx.reshape(1, batch * slots))
    emb = emb.reshape(batch, slots * d)

    w1e = jnp.concatenate(
        [W1[num_dim:], jnp.zeros((slots * d - f * d, h1_dim), jnp.float32)], axis=0
    )
    row = lambda x: x.reshape(1, -1)
    h1, s1 = _layer1(numeric_features, emb, W1[:num_dim], w1e, row(b1))
    h2, s2 = _mid(h1, s1, row(g1), row(be1), W2, row(b2))
    h3, s3 = _mid(h2, s2, row(g2), row(be2), W3, row(b3))
    return _last(h3, s3, row(g3), row(be3), W4, row(b4))


# speedup vs baseline: 7.8181x; 1.4966x over previous
import functools

import jax
import jax.numpy as jnp
from jax import lax
from jax.experimental import pallas as pl
from jax.experimental.pallas import tpu as pltpu
from jax.experimental.pallas import tpu_sc as plsc

EPS = 1e-5
GW = 128
TB = 512
def _transpose_body(x_ref, o_ref):
    d, v = x_ref.shape[1], x_ref.shape[2]
    lanes = 8 * d
    rep = jnp.where(
        lax.broadcasted_iota(jnp.int32, (d, lanes), 1) % d
        == lax.broadcasted_iota(jnp.int32, (d, lanes), 0),
        1.0, 0.0)
    ch = 2048
    chunks = [(i * ch, ch) for i in range(v // ch)]
    if v % ch:
        chunks.append((v - v % ch, v % ch))
    rep = rep.astype(jnp.bfloat16)
    keep = {}
    for off, c in chunks:
        if c not in keep:
            keep[c] = (lax.broadcasted_iota(jnp.int32, (c, lanes), 1) // d
                       == lax.broadcasted_iota(jnp.int32, (c, lanes), 0) % 8)
        x = x_ref[0, :, pl.ds(off, c)].astype(jnp.bfloat16)
        z = lax.dot_general(x, rep, (((0,), (0,)), ((), ())),
                            preferred_element_type=jnp.float32)
        z3 = z.reshape(c // 8, 8, lanes)
        wsel = jnp.broadcast_to(
            (lax.broadcasted_iota(jnp.int32, (1, 1, lanes), 2) // d),
            (c // 8, 1, lanes))
        o_ref[0, pl.ds(off // 8, c // 8), :] = jnp.take_along_axis(
            z3, wsel, axis=1)[:, 0, :]


def _tc_relayout_tables(tables):
    f, v, d = tables.shape
    vg = v // 8
    vgp = ((vg + 7) // 8) * 8
    t_t = jnp.transpose(tables, (0, 2, 1))
    t128 = pl.pallas_call(
        _transpose_body,
        grid=(f,),
        in_specs=[pl.BlockSpec((1, d, v), lambda i: (i, 0, 0))],
        out_specs=pl.BlockSpec((1, vgp, 8 * d), lambda i: (i, 0, 0)),
        out_shape=jax.ShapeDtypeStruct((f, vgp, 8 * d), jnp.float32),
    )(t_t)
    return t128.reshape(f * vgp * 8, d), vgp * 8


def _sc_gather(tables_flat, idx2d):
    n_idx = idx2d.shape[1]
    d = tables_flat.shape[1]
    mesh = plsc.VectorSubcoreMesh(core_axis_name="core", subcore_axis_name="subcore")

    @functools.partial(
        pl.kernel,
        out_type=jax.ShapeDtypeStruct((n_idx, d), tables_flat.dtype),
        mesh=mesh,
        compiler_params=pltpu.CompilerParams(use_tc_tiling_on_sc=False),
    )
    def gather_kernel(tab_hbm, idx_hbm, out_hbm):
        def body(i_vmem, o_vmem):
            pltpu.sync_copy(tab_hbm.at[i_vmem.at[0]], o_vmem)

        pltpu.emit_pipeline(
            body,
            grid=(n_idx // GW,),
            in_specs=[pl.BlockSpec((1, GW), index_map=lambda i: (0, i))],
            out_specs=[pl.BlockSpec((GW, d), index_map=lambda i: (i, 0))],
            core_axis_name=("core", "subcore"),
            dimension_semantics=(pltpu.PARALLEL,),
        )(idx_hbm, out_hbm)

    return gather_kernel(tables_flat, idx2d)


def _stats_update(s_ref, h):
    st = jnp.concatenate(
        [jnp.sum(h, axis=0, keepdims=True), jnp.sum(h * h, axis=0, keepdims=True)], axis=0
    )

    @pl.when(pl.program_id(0) == 0)
    def _():
        s_ref[...] = st

    @pl.when(pl.program_id(0) != 0)
    def _():
        s_ref[...] += st


def _bdot(a, w):
    return jnp.dot(a, w, preferred_element_type=jnp.float32)


def _layer1_body(num_ref, emb_ref, wn_ref, we_ref, b_ref, h_ref, s_ref):
    h = _bdot(num_ref[...], wn_ref[...])
    h = h + _bdot(emb_ref[...], we_ref[...])
    h = h + b_ref[...]
    h_ref[...] = h
    _stats_update(s_ref, h)


def _bn_relu(h, s, g, be, batch):
    mu = s[0:1, :] * (1.0 / batch)
    var = s[1:2, :] * (1.0 / batch) - mu * mu
    scale = g * lax.rsqrt(var + EPS)
    shift = be - mu * scale
    return jnp.maximum(h * scale + shift, 0.0)


def _mid_body(h_ref, s_ref, g_ref, be_ref, w_ref, b_ref, o_ref, so_ref, *, batch):
    a = _bn_relu(h_ref[...], s_ref[...], g_ref[...], be_ref[...], batch)
    h = _bdot(a, w_ref[...]) + b_ref[...]
    o_ref[...] = h
    _stats_update(so_ref, h)


def _last_body(h_ref, s_ref, g_ref, be_ref, w_ref, b_ref, o_ref, *, batch):
    a = _bn_relu(h_ref[...], s_ref[...], g_ref[...], be_ref[...], batch)
    o_ref[...] = _bdot(a, w_ref[...]) + b_ref[...]


def _full(shape):
    return pl.BlockSpec(shape, lambda i: (0,) * len(shape))


def _layer1(num, emb, wn, we, b):
    batch, h1 = num.shape[0], wn.shape[1]
    return pl.pallas_call(
        _layer1_body,
        grid=(batch // TB,),
        in_specs=[
            pl.BlockSpec((TB, num.shape[1]), lambda i: (i, 0)),
            pl.BlockSpec((TB, emb.shape[1]), lambda i: (i, 0)),
            _full(wn.shape),
            _full(we.shape),
            _full(b.shape),
        ],
        out_specs=[
            pl.BlockSpec((TB, h1), lambda i: (i, 0)),
            _full((2, h1)),
        ],
        out_shape=[
            jax.ShapeDtypeStruct((batch, h1), jnp.float32),
            jax.ShapeDtypeStruct((2, h1), jnp.float32),
        ],
    )(num, emb, wn, we, b)


def _mid(h, s, g, be, w, b):
    batch, hout = h.shape[0], w.shape[1]
    return pl.pallas_call(
        functools.partial(_mid_body, batch=batch),
        grid=(batch // TB,),
        in_specs=[
            pl.BlockSpec((TB, h.shape[1]), lambda i: (i, 0)),
            _full(s.shape),
            _full(g.shape),
            _full(be.shape),
            _full(w.shape),
            _full(b.shape),
        ],
        out_specs=[
            pl.BlockSpec((TB, hout), lambda i: (i, 0)),
            _full((2, hout)),
        ],
        out_shape=[
            jax.ShapeDtypeStruct((batch, hout), jnp.float32),
            jax.ShapeDtypeStruct((2, hout), jnp.float32),
        ],
    )(h, s, g, be, w, b)


def _last(h, s, g, be, w, b):
    batch, hout = h.shape[0], w.shape[1]
    return pl.pallas_call(
        functools.partial(_last_body, batch=batch),
        grid=(batch // TB,),
        in_specs=[
            pl.BlockSpec((TB, h.shape[1]), lambda i: (i, 0)),
            _full(s.shape),
            _full(g.shape),
            _full(be.shape),
            _full(w.shape),
            _full(b.shape),
        ],
        out_specs=pl.BlockSpec((TB, hout), lambda i: (i, 0)),
        out_shape=jax.ShapeDtypeStruct((batch, hout), jnp.float32),
    )(h, s, g, be, w, b)


def kernel(numeric_features, cat_features, tables,
           W1, b1, g1, be1, W2, b2, g2, be2, W3, b3, g3, be3, W4, b4):
    batch, num_dim = numeric_features.shape
    f, v, d = tables.shape
    h1_dim = W1.shape[1]

    table_rows, stride = _tc_relayout_tables(tables)

    slots = 32
    idx = cat_features + (jnp.arange(f, dtype=jnp.int32) * stride)[None, :]
    idx = jnp.concatenate([idx, idx[:, : slots - f]], axis=1)
    emb = _sc_gather(table_rows, idx.reshape(1, batch * slots))
    emb = emb.reshape(batch, slots * d)

    w1e = jnp.concatenate(
        [W1[num_dim:], jnp.zeros((slots * d - f * d, h1_dim), jnp.float32)], axis=0
    )
    row = lambda x: x.reshape(1, -1)
    h1, s1 = _layer1(numeric_features, emb, W1[:num_dim], w1e, row(b1))
    h2, s2 = _mid(h1, s1, row(g1), row(be1), W2, row(b2))
    h3, s3 = _mid(h2, s2, row(g2), row(be2), W3, row(b3))
    return _last(h3, s3, row(g3), row(be3), W4, row(b4))


# speedup vs baseline: 8.0311x; 1.0273x over previous
import functools

import jax
import jax.numpy as jnp
from jax import lax
from jax.experimental import pallas as pl
from jax.experimental.pallas import tpu as pltpu
from jax.experimental.pallas import tpu_sc as plsc

EPS = 1e-5
GW = 128
TB = 512
def _transpose_body(x_ref, o_ref):
    d, v = x_ref.shape[1], x_ref.shape[2]
    lanes = 8 * d
    rep = jnp.where(
        lax.broadcasted_iota(jnp.int32, (d, lanes), 1) % d
        == lax.broadcasted_iota(jnp.int32, (d, lanes), 0),
        1.0, 0.0)
    ch = 2048
    chunks = [(i * ch, ch) for i in range(v // ch)]
    if v % ch:
        chunks.append((v - v % ch, v % ch))
    rep = rep.astype(jnp.bfloat16)
    keep = {}
    for off, c in chunks:
        if c not in keep:
            keep[c] = (lax.broadcasted_iota(jnp.int32, (c, lanes), 1) // d
                       == lax.broadcasted_iota(jnp.int32, (c, lanes), 0) % 8)
        x = x_ref[0, :, pl.ds(off, c)].astype(jnp.bfloat16)
        z = lax.dot_general(x, rep, (((0,), (0,)), ((), ())),
                            preferred_element_type=jnp.float32)
        z3 = z.reshape(c // 8, 8, lanes)
        wsel = jnp.broadcast_to(
            (lax.broadcasted_iota(jnp.int32, (1, 1, lanes), 2) // d),
            (c // 8, 1, lanes))
        o_ref[0, pl.ds(off // 8, c // 8), :] = jnp.take_along_axis(
            z3, wsel, axis=1)[:, 0, :]


def _tc_relayout_tables(tables):
    f, v, d = tables.shape
    vg = v // 8
    vgp = ((vg + 7) // 8) * 8
    t_t = jnp.transpose(tables, (0, 2, 1))
    nf = f // 2
    halves = []
    for f0 in (0, nf):
        t128 = pl.pallas_call(
            _transpose_body,
            grid=(nf,),
            in_specs=[pl.BlockSpec((1, d, v), lambda i, f0=f0: (i + f0, 0, 0))],
            out_specs=pl.BlockSpec((1, vgp, 8 * d), lambda i: (i, 0, 0)),
            out_shape=jax.ShapeDtypeStruct((nf, vgp, 8 * d), jnp.float32),
        )(t_t)
        halves.append(t128.reshape(nf * vgp * 8, d))
    return halves, vgp * 8


def _sc_gather(tables_flat, idx2d):
    n_idx = idx2d.shape[1]
    d = tables_flat.shape[1]
    mesh = plsc.VectorSubcoreMesh(core_axis_name="core", subcore_axis_name="subcore")

    @functools.partial(
        pl.kernel,
        out_type=jax.ShapeDtypeStruct((n_idx, d), tables_flat.dtype),
        mesh=mesh,
        compiler_params=pltpu.CompilerParams(use_tc_tiling_on_sc=False),
    )
    def gather_kernel(tab_hbm, idx_hbm, out_hbm):
        def body(i_vmem, o_vmem):
            pltpu.sync_copy(tab_hbm.at[i_vmem.at[0]], o_vmem)

        pltpu.emit_pipeline(
            body,
            grid=(n_idx // GW,),
            in_specs=[pl.BlockSpec((1, GW), index_map=lambda i: (0, i))],
            out_specs=[pl.BlockSpec((GW, d), index_map=lambda i: (i, 0))],
            core_axis_name=("core", "subcore"),
            dimension_semantics=(pltpu.PARALLEL,),
        )(idx_hbm, out_hbm)

    return gather_kernel(tables_flat, idx2d)


def _stats_update(s_ref, h):
    st = jnp.concatenate(
        [jnp.sum(h, axis=0, keepdims=True), jnp.sum(h * h, axis=0, keepdims=True)], axis=0
    )

    @pl.when(pl.program_id(0) == 0)
    def _():
        s_ref[...] = st

    @pl.when(pl.program_id(0) != 0)
    def _():
        s_ref[...] += st


def _bdot(a, w):
    return jnp.dot(a, w, preferred_element_type=jnp.float32)


def _l1a_body(num_ref, emb_ref, wn_ref, we_ref, h_ref):
    h = _bdot(num_ref[...], wn_ref[...])
    h_ref[...] = h + _bdot(emb_ref[...], we_ref[...])


def _l1b_body(hp_ref, emb_ref, we_ref, b_ref, h_ref, s_ref):
    h = hp_ref[...] + _bdot(emb_ref[...], we_ref[...]) + b_ref[...]
    h_ref[...] = h
    _stats_update(s_ref, h)


def _bn_relu(h, s, g, be, batch):
    mu = s[0:1, :] * (1.0 / batch)
    var = s[1:2, :] * (1.0 / batch) - mu * mu
    scale = g * lax.rsqrt(var + EPS)
    shift = be - mu * scale
    return jnp.maximum(h * scale + shift, 0.0)


def _mid_body(h_ref, s_ref, g_ref, be_ref, w_ref, b_ref, o_ref, so_ref, *, batch):
    a = _bn_relu(h_ref[...], s_ref[...], g_ref[...], be_ref[...], batch)
    h = _bdot(a, w_ref[...]) + b_ref[...]
    o_ref[...] = h
    _stats_update(so_ref, h)


def _last_body(h_ref, s_ref, g_ref, be_ref, w_ref, b_ref, o_ref, *, batch):
    a = _bn_relu(h_ref[...], s_ref[...], g_ref[...], be_ref[...], batch)
    o_ref[...] = _bdot(a, w_ref[...]) + b_ref[...]


def _full(shape):
    return pl.BlockSpec(shape, lambda i: (0,) * len(shape))


def _layer1a(num, emb, wn, we):
    batch, h1 = num.shape[0], wn.shape[1]
    return pl.pallas_call(
        _l1a_body,
        grid=(batch // TB,),
        in_specs=[
            pl.BlockSpec((TB, num.shape[1]), lambda i: (i, 0)),
            pl.BlockSpec((TB, emb.shape[1]), lambda i: (i, 0)),
            _full(wn.shape),
            _full(we.shape),
        ],
        out_specs=pl.BlockSpec((TB, h1), lambda i: (i, 0)),
        out_shape=jax.ShapeDtypeStruct((batch, h1), jnp.float32),
    )(num, emb, wn, we)


def _layer1b(hp, emb, we, b):
    batch, h1 = hp.shape
    return pl.pallas_call(
        _l1b_body,
        grid=(batch // TB,),
        in_specs=[
            pl.BlockSpec((TB, h1), lambda i: (i, 0)),
            pl.BlockSpec((TB, emb.shape[1]), lambda i: (i, 0)),
            _full(we.shape),
            _full(b.shape),
        ],
        out_specs=[
            pl.BlockSpec((TB, h1), lambda i: (i, 0)),
            _full((2, h1)),
        ],
        out_shape=[
            jax.ShapeDtypeStruct((batch, h1), jnp.float32),
            jax.ShapeDtypeStruct((2, h1), jnp.float32),
        ],
    )(hp, emb, we, b)


def _mid(h, s, g, be, w, b):
    batch, hout = h.shape[0], w.shape[1]
    return pl.pallas_call(
        functools.partial(_mid_body, batch=batch),
        grid=(batch // TB,),
        in_specs=[
            pl.BlockSpec((TB, h.shape[1]), lambda i: (i, 0)),
            _full(s.shape),
            _full(g.shape),
            _full(be.shape),
            _full(w.shape),
            _full(b.shape),
        ],
        out_specs=[
            pl.BlockSpec((TB, hout), lambda i: (i, 0)),
            _full((2, hout)),
        ],
        out_shape=[
            jax.ShapeDtypeStruct((batch, hout), jnp.float32),
            jax.ShapeDtypeStruct((2, hout), jnp.float32),
        ],
    )(h, s, g, be, w, b)


def _last(h, s, g, be, w, b):
    batch, hout = h.shape[0], w.shape[1]
    return pl.pallas_call(
        functools.partial(_last_body, batch=batch),
        grid=(batch // TB,),
        in_specs=[
            pl.BlockSpec((TB, h.shape[1]), lambda i: (i, 0)),
            _full(s.shape),
            _full(g.shape),
            _full(be.shape),
            _full(w.shape),
            _full(b.shape),
        ],
        out_specs=pl.BlockSpec((TB, hout), lambda i: (i, 0)),
        out_shape=jax.ShapeDtypeStruct((batch, hout), jnp.float32),
    )(h, s, g, be, w, b)


def kernel(numeric_features, cat_features, tables,
           W1, b1, g1, be1, W2, b2, g2, be2, W3, b3, g3, be3, W4, b4):
    batch, num_dim = numeric_features.shape
    f, v, d = tables.shape
    h1_dim = W1.shape[1]

    (table_a, table_b), stride = _tc_relayout_tables(tables)

    nf = f // 2
    slots = 16
    off = (jnp.arange(nf, dtype=jnp.int32) * stride)[None, :]
    idx_a = cat_features[:, :nf] + off
    idx_b = cat_features[:, nf:] + off
    idx_a = jnp.concatenate([idx_a, idx_a[:, : slots - nf]], axis=1)
    idx_b = jnp.concatenate([idx_b, idx_b[:, : slots - nf]], axis=1)
    emb_a = _sc_gather(table_a, idx_a.reshape(1, batch * slots))
    emb_a = emb_a.reshape(batch, slots * d)
    emb_b = _sc_gather(table_b, idx_b.reshape(1, batch * slots))
    emb_b = emb_b.reshape(batch, slots * d)

    zpad = jnp.zeros(((slots - nf) * d, h1_dim), jnp.float32)
    w1e_a = jnp.concatenate([W1[num_dim:num_dim + nf * d], zpad], axis=0)
    w1e_b = jnp.concatenate([W1[num_dim + nf * d:], zpad], axis=0)
    row = lambda x: x.reshape(1, -1)
    h1p = _layer1a(numeric_features, emb_a, W1[:num_dim], w1e_a)
    h1, s1 = _layer1b(h1p, emb_b, w1e_b, row(b1))
    h2, s2 = _mid(h1, s1, row(g1), row(be1), W2, row(b2))
    h3, s3 = _mid(h2, s2, row(g2), row(be2), W3, row(b3))
    return _last(h3, s3, row(g3), row(be3), W4, row(b4))


# speedup vs baseline: 8.0332x; 1.0003x over previous
import functools

import jax
import jax.numpy as jnp
from jax import lax
from jax.experimental import pallas as pl
from jax.experimental.pallas import tpu as pltpu
from jax.experimental.pallas import tpu_sc as plsc

EPS = 1e-5
GW = 128
TB = 512
def _transpose_body(x_ref, o_ref):
    d, v = x_ref.shape[1], x_ref.shape[2]
    lanes = 8 * d
    rep = jnp.where(
        lax.broadcasted_iota(jnp.int32, (d, lanes), 1) % d
        == lax.broadcasted_iota(jnp.int32, (d, lanes), 0),
        1.0, 0.0)
    ch = 2048
    chunks = [(i * ch, ch) for i in range(v // ch)]
    if v % ch:
        chunks.append((v - v % ch, v % ch))
    rep = rep.astype(jnp.bfloat16)
    for off, c in chunks:
        x = x_ref[0, :, pl.ds(off, c)].astype(jnp.bfloat16)
        z = lax.dot_general(x, rep, (((0,), (0,)), ((), ())),
                            preferred_element_type=jnp.float32)
        z3 = z.reshape(c // 8, 8, lanes)
        wsel = jnp.broadcast_to(
            (lax.broadcasted_iota(jnp.int32, (1, 1, lanes), 2) // d),
            (c // 8, 1, lanes))
        o_ref[0, pl.ds(off // 8, c // 8), :] = jnp.take_along_axis(
            z3, wsel, axis=1)[:, 0, :]


def _tc_relayout_tables(tables):
    f, v, d = tables.shape
    vg = v // 8
    vgp = ((vg + 7) // 8) * 8
    t_t = jnp.transpose(tables, (0, 2, 1))
    nf = f // 2
    halves = []
    for f0 in (0, nf):
        t128 = pl.pallas_call(
            _transpose_body,
            grid=(nf,),
            in_specs=[pl.BlockSpec((1, d, v), lambda i, f0=f0: (i + f0, 0, 0))],
            out_specs=pl.BlockSpec((1, vgp, 8 * d), lambda i: (i, 0, 0)),
            out_shape=jax.ShapeDtypeStruct((nf, vgp, 8 * d), jnp.float32),
        )(t_t)
        halves.append(t128.reshape(nf * vgp * 8, d))
    return halves, vgp * 8


def _sc_gather(tables_flat, idx2d):
    n_idx = idx2d.shape[1]
    d = tables_flat.shape[1]
    mesh = plsc.VectorSubcoreMesh(core_axis_name="core", subcore_axis_name="subcore")

    @functools.partial(
        pl.kernel,
        out_type=jax.ShapeDtypeStruct((n_idx, d), tables_flat.dtype),
        mesh=mesh,
        compiler_params=pltpu.CompilerParams(use_tc_tiling_on_sc=False),
    )
    def gather_kernel(tab_hbm, idx_hbm, out_hbm):
        def body(i_vmem, o_vmem):
            pltpu.sync_copy(tab_hbm.at[i_vmem.at[0]], o_vmem)

        pltpu.emit_pipeline(
            body,
            grid=(n_idx // GW,),
            in_specs=[pl.BlockSpec((1, GW), index_map=lambda i: (0, i))],
            out_specs=[pl.BlockSpec((GW, d), index_map=lambda i: (i, 0))],
            core_axis_name=("core", "subcore"),
            dimension_semantics=(pltpu.PARALLEL,),
        )(idx_hbm, out_hbm)

    return gather_kernel(tables_flat, idx2d)


def _stats_update(s_ref, h):
    st = jnp.concatenate(
        [jnp.sum(h, axis=0, keepdims=True), jnp.sum(h * h, axis=0, keepdims=True)], axis=0
    )

    @pl.when(pl.program_id(0) == 0)
    def _():
        s_ref[...] = st

    @pl.when(pl.program_id(0) != 0)
    def _():
        s_ref[...] += st


def _bdot(a, w):
    return jnp.dot(a, w, preferred_element_type=jnp.float32)


def _l1a_body(num_ref, emb_ref, wn_ref, we_ref, h_ref):
    h = _bdot(num_ref[...], wn_ref[...])
    h_ref[...] = h + _bdot(emb_ref[...], we_ref[...])


def _l1b_body(hp_ref, emb_ref, we_ref, b_ref, h_ref, s_ref):
    h = hp_ref[...] + _bdot(emb_ref[...], we_ref[...]) + b_ref[...]
    h_ref[...] = h
    _stats_update(s_ref, h)


def _bn_relu(h, s, g, be, batch):
    mu = s[0:1, :] * (1.0 / batch)
    var = s[1:2, :] * (1.0 / batch) - mu * mu
    scale = g * lax.rsqrt(var + EPS)
    shift = be - mu * scale
    return jnp.maximum(h * scale + shift, 0.0)


def _mid_body(h_ref, s_ref, g_ref, be_ref, w_ref, b_ref, o_ref, so_ref, *, batch):
    a = _bn_relu(h_ref[...], s_ref[...], g_ref[...], be_ref[...], batch)
    h = _bdot(a, w_ref[...]) + b_ref[...]
    o_ref[...] = h
    _stats_update(so_ref, h)


def _last_body(h_ref, s_ref, g_ref, be_ref, w_ref, b_ref, o_ref, *, batch):
    a = _bn_relu(h_ref[...], s_ref[...], g_ref[...], be_ref[...], batch)
    o_ref[...] = _bdot(a, w_ref[...]) + b_ref[...]


def _full(shape):
    return pl.BlockSpec(shape, lambda i: (0,) * len(shape))


def _layer1a(num, emb, wn, we):
    batch, h1 = num.shape[0], wn.shape[1]
    return pl.pallas_call(
        _l1a_body,
        grid=(batch // TB,),
        in_specs=[
            pl.BlockSpec((TB, num.shape[1]), lambda i: (i, 0)),
            pl.BlockSpec((TB, emb.shape[1]), lambda i: (i, 0)),
            _full(wn.shape),
            _full(we.shape),
        ],
        out_specs=pl.BlockSpec((TB, h1), lambda i: (i, 0)),
        out_shape=jax.ShapeDtypeStruct((batch, h1), jnp.float32),
    )(num, emb, wn, we)


def _layer1b(hp, emb, we, b):
    batch, h1 = hp.shape
    return pl.pallas_call(
        _l1b_body,
        grid=(batch // TB,),
        in_specs=[
            pl.BlockSpec((TB, h1), lambda i: (i, 0)),
            pl.BlockSpec((TB, emb.shape[1]), lambda i: (i, 0)),
            _full(we.shape),
            _full(b.shape),
        ],
        out_specs=[
            pl.BlockSpec((TB, h1), lambda i: (i, 0)),
            _full((2, h1)),
        ],
        out_shape=[
            jax.ShapeDtypeStruct((batch, h1), jnp.float32),
            jax.ShapeDtypeStruct((2, h1), jnp.float32),
        ],
    )(hp, emb, we, b)


def _mid(h, s, g, be, w, b):
    batch, hout = h.shape[0], w.shape[1]
    return pl.pallas_call(
        functools.partial(_mid_body, batch=batch),
        grid=(batch // TB,),
        in_specs=[
            pl.BlockSpec((TB, h.shape[1]), lambda i: (i, 0)),
            _full(s.shape),
            _full(g.shape),
            _full(be.shape),
            _full(w.shape),
            _full(b.shape),
        ],
        out_specs=[
            pl.BlockSpec((TB, hout), lambda i: (i, 0)),
            _full((2, hout)),
        ],
        out_shape=[
            jax.ShapeDtypeStruct((batch, hout), jnp.float32),
            jax.ShapeDtypeStruct((2, hout), jnp.float32),
        ],
    )(h, s, g, be, w, b)


def _last(h, s, g, be, w, b):
    batch, hout = h.shape[0], w.shape[1]
    return pl.pallas_call(
        functools.partial(_last_body, batch=batch),
        grid=(batch // TB,),
        in_specs=[
            pl.BlockSpec((TB, h.shape[1]), lambda i: (i, 0)),
            _full(s.shape),
            _full(g.shape),
            _full(be.shape),
            _full(w.shape),
            _full(b.shape),
        ],
        out_specs=pl.BlockSpec((TB, hout), lambda i: (i, 0)),
        out_shape=jax.ShapeDtypeStruct((batch, hout), jnp.float32),
    )(h, s, g, be, w, b)


def kernel(numeric_features, cat_features, tables,
           W1, b1, g1, be1, W2, b2, g2, be2, W3, b3, g3, be3, W4, b4):
    batch, num_dim = numeric_features.shape
    f, v, d = tables.shape
    h1_dim = W1.shape[1]

    (table_a, table_b), stride = _tc_relayout_tables(tables)

    nf = f // 2
    slots = 16
    off = (jnp.arange(nf, dtype=jnp.int32) * stride)[None, :]
    idx_a = cat_features[:, :nf] + off
    idx_b = cat_features[:, nf:] + off
    idx_a = jnp.concatenate([idx_a, idx_a[:, : slots - nf]], axis=1)
    idx_b = jnp.concatenate([idx_b, idx_b[:, : slots - nf]], axis=1)
    emb_a = _sc_gather(table_a, idx_a.reshape(1, batch * slots))
    emb_a = emb_a.reshape(batch, slots * d)
    emb_b = _sc_gather(table_b, idx_b.reshape(1, batch * slots))
    emb_b = emb_b.reshape(batch, slots * d)

    zpad = jnp.zeros(((slots - nf) * d, h1_dim), jnp.float32)
    w1e_a = jnp.concatenate([W1[num_dim:num_dim + nf * d], zpad], axis=0)
    w1e_b = jnp.concatenate([W1[num_dim + nf * d:], zpad], axis=0)
    row = lambda x: x.reshape(1, -1)
    h1p = _layer1a(numeric_features, emb_a, W1[:num_dim], w1e_a)
    h1, s1 = _layer1b(h1p, emb_b, w1e_b, row(b1))
    h2, s2 = _mid(h1, s1, row(g1), row(be1), W2, row(b2))
    h3, s3 = _mid(h2, s2, row(g2), row(be2), W3, row(b3))
    return _last(h3, s3, row(g3), row(be3), W4, row(b4))
